# baseline (speedup 1.0000x reference)
import jax
import jax.numpy as jnp
from jax import lax
from jax.experimental import pallas as pl
from jax.experimental.pallas import tpu as pltpu
from jax.experimental.pallas import tpu_sc as plsc

N = 10000
B = 128
E = 320000
NC, NS, L = 2, 16, 16
W = NC * NS
BLK = 128
BPW = 80
NBLK = BPW * W
E_PAD = NBLK * BLK
STRIPE = 624
STRIPE_LAST = N - STRIPE * (NS - 1)
GRID = 10


def _exp_body(x_ref, o_ref):
    o_ref[...] = jnp.exp(x_ref[...])


def _log_body(p_ref, o_ref):
    s = p_ref[0] + p_ref[1]
    o_ref[...] = jnp.log(jnp.maximum(s, 1e-30))


def _sc_body(ev, src, dst, p, zeros, out,
             src_v, dst_v, p_v, rows_v, s_sh, gsem, ssem, ms0, ms1):
    cid = lax.axis_index("c")
    sid = lax.axis_index("s")
    wid = cid * NS + sid
    ms = (ms0, ms1)

    def meta_block(k_clamped):
        return (wid + k_clamped * W) * BLK

    def issue_meta(e0, slot, sem):
        pltpu.async_copy(src.at[pl.ds(e0, BLK)], src_v.at[slot], sem)
        pltpu.async_copy(dst.at[pl.ds(e0, BLK)], dst_v.at[slot], sem)
        pltpu.async_copy(p.at[pl.ds(e0, BLK)], p_v.at[slot], sem)

    def wait_meta(slot, sem):
        pltpu.make_async_copy(src.at[pl.ds(0, BLK)], src_v.at[slot], sem).wait()
        pltpu.make_async_copy(dst.at[pl.ds(0, BLK)], dst_v.at[slot], sem).wait()
        pltpu.make_async_copy(p.at[pl.ds(0, BLK)], p_v.at[slot], sem).wait()

    e0 = meta_block(0)
    pltpu.sync_copy(src.at[pl.ds(e0, BLK)], src_v.at[0])
    pltpu.sync_copy(dst.at[pl.ds(e0, BLK)], dst_v.at[0])
    pltpu.sync_copy(p.at[pl.ds(e0, BLK)], p_v.at[0])
    issue_meta(meta_block(1), 1, ms1)
    pltpu.async_copy(ev.at[src_v.at[0]], rows_v.at[0], gsem)

    r0 = sid * STRIPE

    @pl.when(sid < NS - 1)
    def _():
        pltpu.sync_copy(zeros.at[pl.ds(r0, STRIPE)],
                        s_sh.at[pl.ds(r0, STRIPE)])

    @pl.when(sid == NS - 1)
    def _():
        pltpu.sync_copy(zeros.at[pl.ds(r0, STRIPE_LAST)],
                        s_sh.at[pl.ds(r0, STRIPE_LAST)])

    plsc.subcore_barrier()

    def outer(t2, carry):
        for u in range(4):
            b = u % 2
            m = u
            m1 = (u + 1) % 4
            m2 = (u + 2) % 4

            pltpu.make_async_copy(ev.at[src_v.at[m]], rows_v.at[b],
                                  gsem).wait()

            def wait_prev_scatter():
                pltpu.make_async_copy(
                    rows_v.at[1 - b], s_sh.at[dst_v.at[(u + 3) % 4]],
                    ssem).wait()

            if u == 0:
                pl.when(t2 >= 1)(wait_prev_scatter)
            else:
                wait_prev_scatter()

            wait_meta(m1, ms[(u + 1) % 2])
            pltpu.async_copy(ev.at[src_v.at[m1]], rows_v.at[1 - b], gsem)

            k2 = 4 * t2 + u + 2
            k2 = jnp.where(k2 < BPW, k2, 0)
            issue_meta(meta_block(k2), m2, ms[u % 2])

            def mul_group(g, c):
                p16 = p_v[m, pl.ds(g * L, L)]
                for k in range(L):
                    ps = jnp.full((L,), p16[k], jnp.float32)
                    row = g * L + k
                    for j in range(B // L):
                        sl = (b, row, pl.ds(j * L, L))
                        rows_v[sl] = rows_v[sl] * ps
                return c

            lax.fori_loop(0, BLK // L, mul_group, 0)

            pltpu.async_copy(rows_v.at[b], s_sh.at[dst_v.at[m]], ssem,
                             add=True)
        return carry

    lax.fori_loop(0, BPW // 4, outer, 0)

    pltpu.make_async_copy(ev.at[src_v.at[0]], rows_v.at[0], gsem).wait()
    wait_meta(1, ms1)
    pltpu.make_async_copy(rows_v.at[1], s_sh.at[dst_v.at[3]], ssem).wait()

    plsc.subcore_barrier()

    @pl.when(sid < NS - 1)
    def _():
        pltpu.sync_copy(s_sh.at[pl.ds(r0, STRIPE)],
                        out.at[cid, pl.ds(r0, STRIPE)])

    @pl.when(sid == NS - 1)
    def _():
        pltpu.sync_copy(s_sh.at[pl.ds(r0, STRIPE_LAST)],
                        out.at[cid, pl.ds(r0, STRIPE_LAST)])


def kernel(ch_vals, edge_src, edge_dst, params):
    ev = pl.pallas_call(
        _exp_body,
        grid=(GRID,),
        in_specs=[pl.BlockSpec((N // GRID, B), lambda i: (i, 0))],
        out_specs=pl.BlockSpec((N // GRID, B), lambda i: (i, 0)),
        out_shape=jax.ShapeDtypeStruct((N, B), jnp.float32),
    )(ch_vals)

    pad = E_PAD - E
    src_p = jnp.concatenate([edge_src, jnp.zeros((pad,), jnp.int32)])
    dst_p = jnp.concatenate([edge_dst, jnp.zeros((pad,), jnp.int32)])
    p_p = jnp.concatenate([params, jnp.zeros((pad,), jnp.float32)])
    zeros = jnp.zeros((N, B), jnp.float32)

    sc = pl.kernel(
        _sc_body,
        out_type=jax.ShapeDtypeStruct((NC, N, B), jnp.float32),
        mesh=plsc.VectorSubcoreMesh(core_axis_name="c", subcore_axis_name="s"),
        scratch_types=[
            pltpu.VMEM((4, BLK), jnp.int32),
            pltpu.VMEM((4, BLK), jnp.int32),
            pltpu.VMEM((4, BLK), jnp.float32),
            pltpu.VMEM((2, BLK, B), jnp.float32),
            pltpu.VMEM_SHARED((N, B), jnp.float32),
            pltpu.SemaphoreType.DMA,
            pltpu.SemaphoreType.DMA,
            pltpu.SemaphoreType.DMA,
            pltpu.SemaphoreType.DMA,
        ],
    )
    partial = sc(ev, src_p, dst_p, p_p, zeros)

    out = pl.pallas_call(
        _log_body,
        grid=(GRID,),
        in_specs=[pl.BlockSpec((NC, N // GRID, B), lambda i: (0, i, 0))],
        out_specs=pl.BlockSpec((N // GRID, B), lambda i: (i, 0)),
        out_shape=jax.ShapeDtypeStruct((N, B), jnp.float32),
    )(partial)
    return out
